# baseline (speedup 1.0000x reference)
import functools

import jax
import jax.numpy as jnp
from jax import lax
from jax.experimental import pallas as pl
from jax.experimental.pallas import tpu as pltpu
from jax.experimental.pallas import tpu_sc as plsc

EMB_ROWS = 100000
D = 768
BATCH = 4
SEQ = 2048
N_TOK = BATCH * SEQ
SCALE = float(D) ** 0.5

_info = plsc.get_sparse_core_info()
NC, NS, L = _info.num_cores, _info.num_subcores, _info.num_lanes
NW = NC * NS
S_PER_W = SEQ // NW
CH = 32
N_CHUNK = BATCH * S_PER_W // CH
GROUPS_PER_ROW = D // L

_mesh = plsc.VectorSubcoreMesh(core_axis_name="c", subcore_axis_name="s")


@functools.partial(
    pl.kernel,
    mesh=_mesh,
    out_type=jax.ShapeDtypeStruct((N_TOK, D), jnp.float32),
    scratch_types=[
        pltpu.VMEM((BATCH * S_PER_W,), jnp.int32),
        pltpu.VMEM((S_PER_W, D), jnp.float32),
        pltpu.VMEM((CH, D), jnp.float32),
        pltpu.VMEM((CH, D), jnp.float32),
        pltpu.VMEM((CH, D), jnp.float32),
        pltpu.SemaphoreType.DMA,
        pltpu.SemaphoreType.DMA,
        pltpu.SemaphoreType.DMA,
        pltpu.SemaphoreType.DMA,
        pltpu.SemaphoreType.DMA,
        pltpu.SemaphoreType.DMA,
        pltpu.SemaphoreType.DMA,
        pltpu.SemaphoreType.DMA,
    ],
)
def _emb_lookup(xr_hbm, emb_hbm, pos_hbm, out_hbm,
                idx_v, pos_v, buf0, buf1, buf2, g0, g1, g2, w0, w1, w2,
                p0, p1):
    wid = lax.axis_index("s") * NC + lax.axis_index("c")
    base = wid * S_PER_W
    bufs = (buf0, buf1, buf2)
    gsems = (g0, g1, g2)
    wsems = (w0, w1, w2)

    pltpu.sync_copy(xr_hbm.at[wid], idx_v)

    def _idx(c):
        return idx_v.at[pl.ds(c * CH, CH)]

    gathers = [None] * N_CHUNK
    writes = [None] * N_CHUNK
    gathers[0] = pltpu.async_copy(emb_hbm.at[_idx(0)], bufs[0], gsems[0])

    pos_cp = [pltpu.async_copy(
        pos_hbm.at[pl.ds(base + h * CH, CH), :],
        pos_v.at[pl.ds(h * CH, CH), :], sem) for h, sem in ((0, p0), (1, p1))]

    for c in range(N_CHUNK):
        cur = c % 3
        nxt = (c + 1) % 3
        gathers[c].wait()
        if c < 2:
            pos_cp[c].wait()
        if c + 1 < N_CHUNK:
            gathers[c + 1] = pltpu.async_copy(
                emb_hbm.at[_idx(c + 1)], bufs[nxt], gsems[nxt])

        b, h = divmod(c, 2)
        buf = bufs[cur]

        def _row_body(i, buf=buf, h=h):
            for j in range(GROUPS_PER_ROW):
                sl = pl.ds(j * L, L)
                buf[i, sl] = buf[i, sl] * SCALE + pos_v[h * CH + i, sl]
        plsc.parallel_loop(0, CH, 1, unroll=2)(_row_body)

        if c >= 1:
            writes[c - 1].wait()
        writes[c] = pltpu.async_copy(
            buf, out_hbm.at[pl.ds(b * SEQ + base + h * CH, CH), :], wsems[cur])

    writes[N_CHUNK - 1].wait()


def kernel(x, emb_weight, pos_weight):
    xr = (x.astype(jnp.int32)
           .reshape(BATCH, NW, S_PER_W)
           .swapaxes(0, 1)
           .reshape(NW, BATCH * S_PER_W))
    out = _emb_lookup(xr, emb_weight, pos_weight)
    return out.reshape(BATCH, SEQ, D)


# speedup vs baseline: 1.1230x; 1.1230x over previous
import functools

import jax
import jax.numpy as jnp
from jax import lax
from jax.experimental import pallas as pl
from jax.experimental.pallas import tpu as pltpu
from jax.experimental.pallas import tpu_sc as plsc

EMB_ROWS = 100000
D = 768
BATCH = 4
SEQ = 2048
N_TOK = BATCH * SEQ
SCALE = float(D) ** 0.5

_info = plsc.get_sparse_core_info()
NC, NS, L = _info.num_cores, _info.num_subcores, _info.num_lanes
NW = NC * NS
S_PER_W = SEQ // NW
CH = 16
N_CHUNK = S_PER_W // CH
GROUPS_PER_ROW = D // L

_mesh = plsc.VectorSubcoreMesh(core_axis_name="c", subcore_axis_name="s")

_BUF = pltpu.VMEM((CH, D), jnp.float32)


@functools.partial(
    pl.kernel,
    mesh=_mesh,
    out_type=jax.ShapeDtypeStruct((N_TOK, D), jnp.float32),
    scratch_types=[
        pltpu.VMEM((BATCH * S_PER_W,), jnp.int32),
        _BUF, _BUF, _BUF, _BUF,
        _BUF,
        _BUF, _BUF, _BUF, _BUF,
        _BUF,
        pltpu.SemaphoreType.DMA,
        pltpu.SemaphoreType.DMA,
        pltpu.SemaphoreType.DMA,
        pltpu.SemaphoreType.DMA,
    ],
)
def _emb_lookup(xr_hbm, emb_hbm, pos_hbm, out_hbm, idx_v,
                a0, a1, a2, a3, ap, b0, b1, b2, b3, bp,
                g0, g1, w0, w1):
    wid = lax.axis_index("s") * NC + lax.axis_index("c")
    base = wid * S_PER_W
    slots = (((a0, a1, a2, a3), ap), ((b0, b1, b2, b3), bp))
    gsems = (g0, g1)
    wsems = (w0, w1)

    pltpu.sync_copy(xr_hbm.at[wid], idx_v)

    def _start_chunk(c, slot):
        bufs, pbuf = slots[slot]
        cps = [pltpu.async_copy(
            pos_hbm.at[pl.ds(base + c * CH, CH), :], pbuf, gsems[slot])]
        for b in range(BATCH):
            cps.append(pltpu.async_copy(
                emb_hbm.at[idx_v.at[pl.ds(b * S_PER_W + c * CH, CH)]],
                bufs[b], gsems[slot]))
        return cps

    gathers = [None] * N_CHUNK
    writes = [None] * N_CHUNK
    gathers[0] = _start_chunk(0, 0)

    for c in range(N_CHUNK):
        cur = c % 2
        nxt = (c + 1) % 2
        for cp in gathers[c]:
            cp.wait()
        if c >= 1:
            for cp in writes[c - 1]:
                cp.wait()
        if c + 1 < N_CHUNK:
            gathers[c + 1] = _start_chunk(c + 1, nxt)

        bufs, pbuf = slots[cur]

        def _row_body(i, bufs=bufs, pbuf=pbuf):
            for j in range(GROUPS_PER_ROW):
                sl = pl.ds(j * L, L)
                pv = pbuf[i, sl]
                for b in range(BATCH):
                    bufs[b][i, sl] = bufs[b][i, sl] * SCALE + pv
        plsc.parallel_loop(0, CH, 1, unroll=2)(_row_body)

        writes[c] = [pltpu.async_copy(
            bufs[b], out_hbm.at[pl.ds(b * SEQ + base + c * CH, CH), :],
            wsems[cur]) for b in range(BATCH)]

    for cp in writes[N_CHUNK - 1]:
        cp.wait()


def kernel(x, emb_weight, pos_weight):
    xr = (x.astype(jnp.int32)
           .reshape(BATCH, NW, S_PER_W)
           .swapaxes(0, 1)
           .reshape(NW, BATCH * S_PER_W))
    out = _emb_lookup(xr, emb_weight, pos_weight)
    return out.reshape(BATCH, SEQ, D)
